# baseline (speedup 1.0000x reference)
import functools

import jax
import jax.numpy as jnp
from jax import lax
from jax.experimental import pallas as pl
from jax.experimental.pallas import tpu as pltpu
from jax.experimental.pallas import tpu_sc as plsc

NC = 2
NS = 16
NW = NC * NS
CHUNK = 8
NBUF = 7
KSC = 4


def _build_normalize(vocab, dim):
    blk = 1024

    def norm_kernel(w_ref, o_ref):
        v = w_ref[...]
        ss = jnp.sum(v * v, axis=1, keepdims=True)
        o_ref[...] = v * lax.rsqrt(jnp.maximum(ss, 1e-24))

    return pl.pallas_call(
        norm_kernel,
        grid=(vocab // blk,),
        in_specs=[pl.BlockSpec((blk, dim), lambda i: (i, 0))],
        out_specs=pl.BlockSpec((blk, dim), lambda i: (i, 0)),
        out_shape=jax.ShapeDtypeStruct((vocab, dim), jnp.float32),
    )


def _build_gather(rows, dim):
    per_w = rows // NW
    nchunk = per_w // CHUNK
    mesh = plsc.VectorSubcoreMesh(core_axis_name="c", subcore_axis_name="s")

    @functools.partial(
        pl.kernel,
        mesh=mesh,
        out_type=jax.ShapeDtypeStruct((rows, dim), jnp.float32),
        scratch_types=(
            [pltpu.VMEM((nchunk, CHUNK), jnp.int32)]
            + [pltpu.VMEM((CHUNK, dim), jnp.float32)] * NBUF
            + [pltpu.SemaphoreType.DMA] * (2 * NBUF)
        ),
    )
    def gather_kernel(x_hbm, table_hbm, out_hbm, idx_v, *scr):
        bufs = scr[:NBUF]
        gsems = scr[NBUF:2 * NBUF]
        wsems = scr[2 * NBUF:]
        cid = lax.axis_index("c")
        sid = lax.axis_index("s")
        wid = sid * NC + cid
        base = wid * per_w
        n = nchunk

        pltpu.sync_copy(x_hbm.at[wid], idx_v)

        def start_gather(g, b):
            pltpu.async_copy(table_hbm.at[idx_v.at[g]], bufs[b], gsems[b])

        def wait_gather(b):
            pltpu.make_async_copy(
                out_hbm.at[pl.ds(base, CHUNK)], bufs[b], gsems[b]
            ).wait()

        def start_scatter(g, b):
            pltpu.async_copy(
                bufs[b], out_hbm.at[pl.ds(base + g * CHUNK, CHUNK)], wsems[b]
            )

        def wait_scatter(b):
            pltpu.make_async_copy(
                bufs[b], out_hbm.at[pl.ds(base, CHUNK)], wsems[b]
            ).wait()

        def step(g, b, first_round=False):
            wait_gather(b)
            start_scatter(g, b)
            if not first_round:
                wait_scatter((b + NBUF - KSC) % NBUF)

            @pl.when(g + NBUF - KSC < n)
            def _():
                start_gather(g + NBUF - KSC, (b + NBUF - KSC) % NBUF)

        for j in range(NBUF - KSC):
            start_gather(j, j)
        for g in range(NBUF):
            step(g, g, first_round=(g < KSC))

        def core(i, carry):
            g0 = i * NBUF
            for b in range(NBUF):
                step(g0 + b, b)
            return carry

        lax.fori_loop(1, n // NBUF, core, 0)

        for g in range((n // NBUF) * NBUF, n):
            step(g, g % NBUF)
        for g in range(n - KSC, n):
            wait_scatter(g % NBUF)

    return gather_kernel


_CACHE = {}


def kernel(x, emb_weight):
    b, s = x.shape
    vocab, dim = emb_weight.shape
    rows = b * s
    key = (rows, vocab, dim)
    if key not in _CACHE:
        _CACHE[key] = (_build_normalize(vocab, dim), _build_gather(rows, dim))
    normalize, gather = _CACHE[key]
    table_n = normalize(emb_weight)
    xw = x.reshape(NW, rows // (NW * CHUNK), CHUNK).astype(jnp.int32)
    out = gather(xw, table_n)
    return out.reshape(b, s, dim)


# speedup vs baseline: 1.0040x; 1.0040x over previous
import functools

import jax
import jax.numpy as jnp
from jax import lax
from jax.experimental import pallas as pl
from jax.experimental.pallas import tpu as pltpu
from jax.experimental.pallas import tpu_sc as plsc

NC = 2
NS = 16
NW = NC * NS
CHUNK = 8
NBUF = 6
KSC = 4


def _build_normalize(vocab, dim):
    blk = 1024

    def norm_kernel(w_ref, o_ref):
        v = w_ref[...]
        ss = jnp.sum(v * v, axis=1, keepdims=True)
        o_ref[...] = v * lax.rsqrt(jnp.maximum(ss, 1e-24))

    return pl.pallas_call(
        norm_kernel,
        grid=(vocab // blk,),
        in_specs=[pl.BlockSpec((blk, dim), lambda i: (i, 0))],
        out_specs=pl.BlockSpec((blk, dim), lambda i: (i, 0)),
        out_shape=jax.ShapeDtypeStruct((vocab, dim), jnp.float32),
    )


def _build_gather(rows, dim):
    per_w = rows // NW
    nchunk = per_w // CHUNK
    mesh = plsc.VectorSubcoreMesh(core_axis_name="c", subcore_axis_name="s")

    @functools.partial(
        pl.kernel,
        mesh=mesh,
        out_type=jax.ShapeDtypeStruct((rows, dim), jnp.float32),
        scratch_types=(
            [pltpu.VMEM((nchunk, CHUNK), jnp.int32)]
            + [pltpu.VMEM((CHUNK, dim), jnp.float32)] * NBUF
            + [pltpu.SemaphoreType.DMA] * (2 * NBUF)
        ),
    )
    def gather_kernel(x_hbm, table_hbm, out_hbm, idx_v, *scr):
        bufs = scr[:NBUF]
        gsems = scr[NBUF:2 * NBUF]
        wsems = scr[2 * NBUF:]
        cid = lax.axis_index("c")
        sid = lax.axis_index("s")
        wid = sid * NC + cid
        base = wid * per_w
        n = nchunk

        pltpu.sync_copy(x_hbm.at[wid], idx_v)

        def start_gather(g, b):
            pltpu.async_copy(table_hbm.at[idx_v.at[g]], bufs[b], gsems[b])

        def wait_gather(b):
            pltpu.make_async_copy(
                out_hbm.at[pl.ds(base, CHUNK)], bufs[b], gsems[b]
            ).wait()

        def start_scatter(g, b):
            pltpu.async_copy(
                bufs[b], out_hbm.at[pl.ds(base + g * CHUNK, CHUNK)], wsems[b]
            )

        def wait_scatter(b):
            pltpu.make_async_copy(
                bufs[b], out_hbm.at[pl.ds(base, CHUNK)], wsems[b]
            ).wait()

        def step(g, b, first_round=False):
            wait_gather(b)
            start_scatter(g, b)
            if not first_round:
                wait_scatter((b + NBUF - KSC) % NBUF)

            @pl.when(g + NBUF - KSC < n)
            def _():
                start_gather(g + NBUF - KSC, (b + NBUF - KSC) % NBUF)

        for j in range(NBUF - KSC):
            start_gather(j, j)
        for g in range(NBUF):
            step(g, g, first_round=(g < KSC))

        def core(i, carry):
            g0 = i * NBUF
            for b in range(NBUF):
                step(g0 + b, b)
            return carry

        lax.fori_loop(1, n // NBUF, core, 0)

        for g in range((n // NBUF) * NBUF, n):
            step(g, g % NBUF)
        for g in range(n - KSC, n):
            wait_scatter(g % NBUF)

    return gather_kernel


_CACHE = {}


def kernel(x, emb_weight):
    b, s = x.shape
    vocab, dim = emb_weight.shape
    rows = b * s
    key = (rows, vocab, dim)
    if key not in _CACHE:
        _CACHE[key] = (_build_normalize(vocab, dim), _build_gather(rows, dim))
    normalize, gather = _CACHE[key]
    table_n = normalize(emb_weight)
    xw = x.reshape(NW, rows // (NW * CHUNK), CHUNK).astype(jnp.int32)
    out = gather(xw, table_n)
    return out.reshape(b, s, dim)
